# baseline (speedup 1.0000x reference)
import functools

import jax
import jax.numpy as jnp
from jax import lax
from jax.experimental import pallas as pl
from jax.experimental.pallas import tpu as pltpu
from jax.experimental.pallas import tpu_sc as plsc


def _make_flatten(V, D, NC, NS):
    NW = NC * NS
    CB = 400
    assert V % CB == 0
    n_ch = V // CB

    mesh = plsc.VectorSubcoreMesh(core_axis_name="c", subcore_axis_name="s")

    @functools.partial(
        pl.kernel,
        mesh=mesh,
        out_type=jax.ShapeDtypeStruct((V, 2 * D), jnp.float32),
        scratch_types=[
            pltpu.VMEM((CB, D), jnp.float32),
            pltpu.VMEM((CB, 2 * D), jnp.float32),
            pltpu.SemaphoreType.DMA,
        ],
        compiler_params=pltpu.CompilerParams(use_tc_tiling_on_sc=True),
    )
    def flatten_kernel(table_hbm, flat_hbm, v64, v128, sem):
        wid = lax.axis_index("s") * NC + lax.axis_index("c")
        n_mine = (n_ch - wid + NW - 1) // NW

        def chunk(k, carry):
            b = (wid + k * NW) * CB
            pltpu.sync_copy(table_hbm.at[pl.ds(b, CB)], v64)

            def rowcopy(i, c2):
                for c in range(D // 16):
                    v128[i, pl.ds(16 * c, 16)] = v64[i, pl.ds(16 * c, 16)]
                return c2

            lax.fori_loop(0, CB, rowcopy, 0)
            pltpu.async_copy(v128, flat_hbm.at[pl.ds(b, CB)], sem).wait()
            return carry

        lax.fori_loop(0, n_mine, chunk, 0)

    return flatten_kernel


def _make_lookup(B, F, V, D, NC, NS):
    NW = NC * NS
    assert B % NW == 0
    rows_w = B // NW
    R = 8
    assert rows_w % R == 0
    n_ch = rows_w // R

    mesh = plsc.VectorSubcoreMesh(core_axis_name="c", subcore_axis_name="s")

    @functools.partial(
        pl.kernel,
        mesh=mesh,
        out_type=jax.ShapeDtypeStruct((B, F, D), jnp.float32),
        scratch_types=[
            pltpu.VMEM((R, F), jnp.int32),
            pltpu.VMEM((R, F, 2 * D), jnp.float32),
            pltpu.VMEM((R, F, D), jnp.float32),
            pltpu.SemaphoreType.DMA,
            pltpu.SemaphoreType.DMA,
        ],
        compiler_params=pltpu.CompilerParams(use_tc_tiling_on_sc=True),
    )
    def lookup_kernel(
        x_hbm, flat_hbm, out_hbm, idx_v, rows_v, dense_v, gsem, osem
    ):
        wid = lax.axis_index("s") * NC + lax.axis_index("c")
        base = wid * rows_w

        def chunk(g, carry):
            r0 = base + g * R
            pltpu.sync_copy(x_hbm.at[pl.ds(r0, R)], idx_v)
            for r in range(R):
                pltpu.async_copy(
                    flat_hbm.at[idx_v.at[r]], rows_v.at[r], gsem
                )
            for r in range(R):
                pltpu.make_async_copy(
                    flat_hbm.at[idx_v.at[r]], rows_v.at[r], gsem
                ).wait()
            def compact(r, c3):
                for f in range(F):
                    for c in range(D // 16):
                        dense_v[r, f, pl.ds(16 * c, 16)] = rows_v[
                            r, f, pl.ds(16 * c, 16)
                        ]
                return c3

            lax.fori_loop(0, R, compact, 0)
            for r in range(R):
                pltpu.async_copy(dense_v.at[r], out_hbm.at[r0 + r], osem)
            for r in range(R):
                pltpu.make_async_copy(
                    dense_v.at[r], out_hbm.at[r0 + r], osem
                ).wait()
            return carry

        lax.fori_loop(0, n_ch, chunk, 0)

    return lookup_kernel


def kernel(x, weight):
    B, F = x.shape
    V, D = weight.shape
    info = plsc.get_sparse_core_info()
    NC, NS = info.num_cores, info.num_subcores
    flat = _make_flatten(V, D, NC, NS)(weight)
    return _make_lookup(B, F, V, D, NC, NS)(x, flat)
